# baseline (speedup 1.0000x reference)
import functools

import jax
import jax.numpy as jnp
from jax import lax
from jax.experimental import pallas as pl
from jax.experimental.pallas import tpu as pltpu
from jax.experimental.pallas import tpu_sc as plsc

N = 10000
E = 160000
D = 256
DH = D // 2
NC, NS, L = 2, 16, 16
K = 80
NCHUNK = (E // NS) // K
BLK = 1000
EPS = 1e-5

_sc_mesh = plsc.VectorSubcoreMesh(
    core_axis_name="c", subcore_axis_name="s", num_cores=NC, num_subcores=NS)


def _degree_body(eidx, deg_hbm, idx_v, ones_v, zb_v, acc_sh):
    c = lax.axis_index("c")
    s = lax.axis_index("s")
    pltpu.sync_copy(eidx.at[c, s], idx_v)

    @pl.loop(0, K // L)
    def _(i):
        ones_v[pl.ds(i * L, L)] = jnp.ones((L,), jnp.float32)

    @pl.when(s == 0)
    def _():
        @pl.loop(0, N // L)
        def _(i):
            zb_v[pl.ds(i * L, L)] = jnp.zeros((L,), jnp.float32)
        pltpu.sync_copy(zb_v, acc_sh)

    plsc.subcore_barrier()

    @pl.loop(0, NCHUNK)
    def _(j):
        pltpu.sync_copy(ones_v, acc_sh.at[idx_v.at[j]], add=True)

    plsc.subcore_barrier()

    @pl.when(s == 0)
    def _():
        pltpu.sync_copy(acc_sh, zb_v)
        pltpu.sync_copy(zb_v, deg_hbm.at[pl.ds(c * N, N)])


_degree_kernel = functools.partial(
    pl.kernel,
    out_type=jax.ShapeDtypeStruct((NC * N,), jnp.float32),
    mesh=_sc_mesh,
    scratch_types=[
        pltpu.VMEM((NCHUNK, K), jnp.int32),
        pltpu.VMEM((K,), jnp.float32),
        pltpu.VMEM((N,), jnp.float32),
        pltpu.VMEM_SHARED((N,), jnp.float32),
    ],
)(_degree_body)


def _matmul_body(x_ref, w_ref, deg_ref, out_ref):
    nsrc = lax.rsqrt(jnp.maximum(deg_ref[...], 1.0))
    h = jnp.dot(x_ref[...], w_ref[...],
                preferred_element_type=jnp.float32)
    h = h * nsrc
    out_ref[0] = h[:, :DH]
    out_ref[1] = h[:, DH:]


def _matmul_call(x, w, deg_out_col):
    return pl.pallas_call(
        _matmul_body,
        grid=(N // BLK,),
        in_specs=[
            pl.BlockSpec((BLK, D), lambda i: (i, 0)),
            pl.BlockSpec((D, D), lambda i: (0, 0)),
            pl.BlockSpec((BLK, 1), lambda i: (i, 0)),
        ],
        out_specs=pl.BlockSpec((2, BLK, DH), lambda i: (0, i, 0)),
        out_shape=jax.ShapeDtypeStruct((2, N, DH), jnp.float32),
    )(x, w, deg_out_col)


CK = 80
SCH = 32
EP = 10240
NCH = EP // CK
NPH = NCH // SCH
TRASH = N
WCH = 40
ACC_R = N + WCH
NZCH = ACC_R // WCH
NWCH = N // WCH


def _agg_body(srcmod, dst4, h2, out_hbm, sidx_v, didx_v, rows0, rows1, zb_v,
              wb_v, acc_sh, sem0, sem1):
    c = lax.axis_index("c")
    s = lax.axis_index("s")

    @pl.loop(0, WCH)
    def _(i):
        for k in range(DH // L):
            zb_v[i, pl.ds(k * L, L)] = jnp.zeros((L,), jnp.float32)

    @pl.loop(s, NZCH, step=NS)
    def _(z):
        pltpu.sync_copy(zb_v, acc_sh.at[pl.ds(z * WCH, WCH)])

    plsc.subcore_barrier()

    @pl.loop(0, NPH)
    def _(ph):
        o = pl.multiple_of(ph * SCH, 8)
        pltpu.sync_copy(srcmod.at[c, s, pl.ds(o, SCH)], sidx_v)
        pltpu.sync_copy(dst4.at[s, pl.ds(o, SCH)], didx_v)

        pltpu.async_copy(h2.at[sidx_v.at[0]], rows0, sem0)

        @pl.loop(0, SCH // 2)
        def _(g):
            j0 = 2 * g
            pltpu.make_async_copy(h2.at[sidx_v.at[j0]], rows0, sem0).wait()
            pltpu.async_copy(h2.at[sidx_v.at[j0 + 1]], rows1, sem1)
            pltpu.sync_copy(rows0, acc_sh.at[didx_v.at[j0]], add=True)
            pltpu.make_async_copy(
                h2.at[sidx_v.at[j0 + 1]], rows1, sem1).wait()

            @pl.when(j0 + 2 < SCH)
            def _():
                pltpu.async_copy(h2.at[sidx_v.at[j0 + 2]], rows0, sem0)

            pltpu.sync_copy(rows1, acc_sh.at[didx_v.at[j0 + 1]], add=True)

    plsc.subcore_barrier()

    @pl.loop(s, NWCH, step=NS)
    def _(z):
        pltpu.sync_copy(acc_sh.at[pl.ds(z * WCH, WCH)], wb_v)
        pltpu.sync_copy(wb_v, out_hbm.at[c, pl.ds(z * WCH, WCH)])


_agg_kernel = functools.partial(
    pl.kernel,
    out_type=jax.ShapeDtypeStruct((NC, N, DH), jnp.float32),
    mesh=_sc_mesh,
    scratch_types=[
        pltpu.VMEM((SCH, CK), jnp.int32),
        pltpu.VMEM((SCH, CK), jnp.int32),
        pltpu.VMEM((CK, DH), jnp.float32),
        pltpu.VMEM((CK, DH), jnp.float32),
        pltpu.VMEM((WCH, DH), jnp.float32),
        pltpu.VMEM((WCH, DH), jnp.float32),
        pltpu.VMEM_SHARED((ACC_R, DH), jnp.float32),
        pltpu.SemaphoreType.DMA,
        pltpu.SemaphoreType.DMA,
    ],
)(_agg_body)


def _epilogue_body(agg_ref, deg_ref, b_ref, g_ref, be_ref, out_ref):
    ndst = lax.rsqrt(jnp.maximum(deg_ref[...], 1.0))
    inv = jnp.float32(1.0) / jnp.sqrt(jnp.float32(1.0 + EPS))
    for q in range(2):
        cols = slice(q * DH, (q + 1) * DH)
        y = agg_ref[q] * ndst + b_ref[:, cols]
        y = y * (g_ref[:, cols] * inv) + be_ref[:, cols]
        out_ref[:, cols] = jnp.maximum(y, 0.0)


def _epilogue_call(agg2, deg_in_col, b2, g2, be2):
    return pl.pallas_call(
        _epilogue_body,
        grid=(N // BLK,),
        in_specs=[
            pl.BlockSpec((2, BLK, DH), lambda i: (0, i, 0)),
            pl.BlockSpec((BLK, 1), lambda i: (i, 0)),
            pl.BlockSpec((1, D), lambda i: (0, 0)),
            pl.BlockSpec((1, D), lambda i: (0, 0)),
            pl.BlockSpec((1, D), lambda i: (0, 0)),
        ],
        out_specs=pl.BlockSpec((BLK, D), lambda i: (i, 0)),
        out_shape=jax.ShapeDtypeStruct((N, D), jnp.float32),
    )(agg2, deg_in_col, b2, g2, be2)


def kernel(node_features, edge_index, W, b, gamma, beta):
    src = edge_index[0]
    dst = edge_index[1]
    eidx_a = edge_index.reshape(NC, NS, NCHUNK, K)
    ept = E // NS
    srcp = jnp.pad(src.reshape(NS, ept), ((0, 0), (0, EP - ept)))
    dstp = jnp.pad(dst.reshape(NS, ept), ((0, 0), (0, EP - ept)),
                   constant_values=TRASH)
    srcmod = jnp.stack([srcp, srcp + N]).reshape(NC, NS, NCH, CK)
    dst4 = dstp.reshape(NS, NCH, CK)

    degs = _degree_kernel(eidx_a)
    deg_out_col = degs[:N].reshape(N, 1)
    deg_in_col = degs[N:].reshape(N, 1)

    h2 = _matmul_call(node_features, W, deg_out_col)

    agg2 = _agg_kernel(srcmod, dst4, h2.reshape(2 * N, DH))

    return _epilogue_call(agg2, deg_in_col, b.reshape(1, D),
                          gamma.reshape(1, D), beta.reshape(1, D))


# speedup vs baseline: 1.0571x; 1.0571x over previous
import functools

import jax
import jax.numpy as jnp
from jax import lax
from jax.experimental import pallas as pl
from jax.experimental.pallas import tpu as pltpu
from jax.experimental.pallas import tpu_sc as plsc

N = 10000
E = 160000
D = 256
DH = D // 2
NC, NS, L = 2, 16, 16
KA = 128
NCHA = 80
BLK = 1000
EPS = 1e-5

_sc_mesh = plsc.VectorSubcoreMesh(
    core_axis_name="c", subcore_axis_name="s", num_cores=NC, num_subcores=NS)


def _degree_body(eidx, deg_hbm, idx_v, ones_v, zb_v, acc_sh, sem):
    c = lax.axis_index("c")
    s = lax.axis_index("s")
    pltpu.sync_copy(eidx.at[c, s], idx_v)

    @pl.loop(0, KA // L)
    def _(i):
        ones_v[pl.ds(i * L, L)] = jnp.ones((L,), jnp.float32)

    @pl.when(s == 0)
    def _():
        @pl.loop(0, N // L)
        def _(i):
            zb_v[pl.ds(i * L, L)] = jnp.zeros((L,), jnp.float32)
        pltpu.sync_copy(zb_v, acc_sh.at[pl.ds(0, N)])
        pltpu.sync_copy(zb_v.at[pl.ds(0, 16)], acc_sh.at[pl.ds(N, 16)])

    plsc.subcore_barrier()

    @pl.loop(0, NCHA)
    def _(j):
        pltpu.async_copy(ones_v, acc_sh.at[idx_v.at[j]], sem, add=True)

    @pl.loop(0, NCHA)
    def _(j):
        pltpu.make_async_copy(ones_v, acc_sh.at[idx_v.at[0]], sem).wait()

    plsc.subcore_barrier()

    @pl.when(s == 0)
    def _():
        pltpu.sync_copy(acc_sh.at[pl.ds(0, N)], zb_v)
        pltpu.sync_copy(zb_v, deg_hbm.at[pl.ds(c * N, N)])


_degree_kernel = functools.partial(
    pl.kernel,
    out_type=jax.ShapeDtypeStruct((NC * N,), jnp.float32),
    mesh=_sc_mesh,
    scratch_types=[
        pltpu.VMEM((NCHA, KA), jnp.int32),
        pltpu.VMEM((KA,), jnp.float32),
        pltpu.VMEM((N,), jnp.float32),
        pltpu.VMEM_SHARED((N + 16,), jnp.float32),
        pltpu.SemaphoreType.DMA,
    ],
)(_degree_body)


def _matmul_body(x_ref, w_ref, deg_ref, out_ref):
    nsrc = lax.rsqrt(jnp.maximum(deg_ref[...], 1.0))
    h = jnp.dot(x_ref[...], w_ref[...],
                preferred_element_type=jnp.float32)
    h = h * nsrc
    out_ref[0] = h[:, :DH]
    out_ref[1] = h[:, DH:]


def _matmul_call(x, w, deg_out_col):
    return pl.pallas_call(
        _matmul_body,
        grid=(N // BLK,),
        in_specs=[
            pl.BlockSpec((BLK, D), lambda i: (i, 0)),
            pl.BlockSpec((D, D), lambda i: (0, 0)),
            pl.BlockSpec((BLK, 1), lambda i: (i, 0)),
        ],
        out_specs=pl.BlockSpec((2, BLK, DH), lambda i: (0, i, 0)),
        out_shape=jax.ShapeDtypeStruct((2, N, DH), jnp.float32),
    )(x, w, deg_out_col)


CK = 128
SCH = 16
EP = 10240
NCH = EP // CK
NPH = NCH // SCH
TRASH = N
WCH = 40
ACC_R = N + WCH
NZCH = ACC_R // WCH
NWCH = N // WCH


def _agg_body(srcmod, dst4, h2, out_hbm, sidx_v, didx_v, rows0, rows1, zb_v,
              wb_v, acc_sh, sem0, sem1):
    c = lax.axis_index("c")
    s = lax.axis_index("s")

    @pl.loop(0, WCH)
    def _(i):
        for k in range(DH // L):
            zb_v[i, pl.ds(k * L, L)] = jnp.zeros((L,), jnp.float32)

    @pl.loop(s, NZCH, step=NS)
    def _(z):
        pltpu.sync_copy(zb_v, acc_sh.at[pl.ds(z * WCH, WCH)])

    plsc.subcore_barrier()

    @pl.loop(0, NPH)
    def _(ph):
        o = pl.multiple_of(ph * SCH, 8)
        pltpu.sync_copy(srcmod.at[c, s, pl.ds(o, SCH)], sidx_v)
        pltpu.sync_copy(dst4.at[s, pl.ds(o, SCH)], didx_v)

        pltpu.async_copy(h2.at[sidx_v.at[0]], rows0, sem0)

        @pl.loop(0, SCH // 2)
        def _(g):
            j0 = 2 * g
            pltpu.make_async_copy(h2.at[sidx_v.at[j0]], rows0, sem0).wait()
            pltpu.async_copy(h2.at[sidx_v.at[j0 + 1]], rows1, sem1)
            pltpu.sync_copy(rows0, acc_sh.at[didx_v.at[j0]], add=True)
            pltpu.make_async_copy(
                h2.at[sidx_v.at[j0 + 1]], rows1, sem1).wait()

            @pl.when(j0 + 2 < SCH)
            def _():
                pltpu.async_copy(h2.at[sidx_v.at[j0 + 2]], rows0, sem0)

            pltpu.sync_copy(rows1, acc_sh.at[didx_v.at[j0 + 1]], add=True)

    plsc.subcore_barrier()

    @pl.loop(s, NWCH, step=NS)
    def _(z):
        pltpu.sync_copy(acc_sh.at[pl.ds(z * WCH, WCH)], wb_v)
        pltpu.sync_copy(wb_v, out_hbm.at[c, pl.ds(z * WCH, WCH)])


_agg_kernel = functools.partial(
    pl.kernel,
    out_type=jax.ShapeDtypeStruct((NC, N, DH), jnp.float32),
    mesh=_sc_mesh,
    scratch_types=[
        pltpu.VMEM((SCH, CK), jnp.int32),
        pltpu.VMEM((SCH, CK), jnp.int32),
        pltpu.VMEM((CK, DH), jnp.float32),
        pltpu.VMEM((CK, DH), jnp.float32),
        pltpu.VMEM((WCH, DH), jnp.float32),
        pltpu.VMEM((WCH, DH), jnp.float32),
        pltpu.VMEM_SHARED((ACC_R, DH), jnp.float32),
        pltpu.SemaphoreType.DMA,
        pltpu.SemaphoreType.DMA,
    ],
)(_agg_body)


def _epilogue_body(agg_ref, deg_ref, b_ref, g_ref, be_ref, out_ref):
    ndst = lax.rsqrt(jnp.maximum(deg_ref[...], 1.0))
    inv = jnp.float32(1.0) / jnp.sqrt(jnp.float32(1.0 + EPS))
    for q in range(2):
        cols = slice(q * DH, (q + 1) * DH)
        y = agg_ref[q] * ndst + b_ref[:, cols]
        y = y * (g_ref[:, cols] * inv) + be_ref[:, cols]
        out_ref[:, cols] = jnp.maximum(y, 0.0)


def _epilogue_call(agg2, deg_in_col, b2, g2, be2):
    return pl.pallas_call(
        _epilogue_body,
        grid=(N // BLK,),
        in_specs=[
            pl.BlockSpec((2, BLK, DH), lambda i: (0, i, 0)),
            pl.BlockSpec((BLK, 1), lambda i: (i, 0)),
            pl.BlockSpec((1, D), lambda i: (0, 0)),
            pl.BlockSpec((1, D), lambda i: (0, 0)),
            pl.BlockSpec((1, D), lambda i: (0, 0)),
        ],
        out_specs=pl.BlockSpec((BLK, D), lambda i: (i, 0)),
        out_shape=jax.ShapeDtypeStruct((N, D), jnp.float32),
    )(agg2, deg_in_col, b2, g2, be2)


def kernel(node_features, edge_index, W, b, gamma, beta):
    src = edge_index[0]
    dst = edge_index[1]
    ept = E // NS
    srcp = jnp.pad(src.reshape(NS, ept), ((0, 0), (0, EP - ept)))
    dstp = jnp.pad(dst.reshape(NS, ept), ((0, 0), (0, EP - ept)),
                   constant_values=TRASH)
    srcpa = jnp.pad(src.reshape(NS, ept), ((0, 0), (0, EP - ept)),
                    constant_values=N)
    eidx_a = jnp.stack([srcpa, dstp]).reshape(NC, NS, NCHA, KA)
    srcmod = jnp.stack([srcp, srcp + N]).reshape(NC, NS, NCH, CK)
    dst4 = dstp.reshape(NS, NCH, CK)

    degs = _degree_kernel(eidx_a)
    deg_out_col = degs[:N].reshape(N, 1)
    deg_in_col = degs[N:].reshape(N, 1)

    h2 = _matmul_call(node_features, W, deg_out_col)

    agg2 = _agg_kernel(srcmod, dst4, h2.reshape(2 * N, DH))

    return _epilogue_call(agg2, deg_in_col, b.reshape(1, D),
                          gamma.reshape(1, D), beta.reshape(1, D))


# speedup vs baseline: 1.0991x; 1.0398x over previous
import functools

import jax
import jax.numpy as jnp
from jax import lax
from jax.experimental import pallas as pl
from jax.experimental.pallas import tpu as pltpu
from jax.experimental.pallas import tpu_sc as plsc

N = 10000
E = 160000
D = 256
DH = D // 2
NC, NS, L = 2, 16, 16
KA = 128
NCHA = 80
BLK = 1000
EPS = 1e-5

_sc_mesh = plsc.VectorSubcoreMesh(
    core_axis_name="c", subcore_axis_name="s", num_cores=NC, num_subcores=NS)


def _degree_body(eidx, deg_hbm, idx_v, ones_v, zb_v, acc_sh, sem):
    c = lax.axis_index("c")
    s = lax.axis_index("s")
    pltpu.sync_copy(eidx.at[c, s], idx_v)

    @pl.loop(0, KA // L)
    def _(i):
        ones_v[pl.ds(i * L, L)] = jnp.ones((L,), jnp.float32)

    @pl.when(s == 0)
    def _():
        @pl.loop(0, N // L)
        def _(i):
            zb_v[pl.ds(i * L, L)] = jnp.zeros((L,), jnp.float32)
        pltpu.sync_copy(zb_v, acc_sh.at[pl.ds(0, N)])
        pltpu.sync_copy(zb_v.at[pl.ds(0, 16)], acc_sh.at[pl.ds(N, 16)])

    plsc.subcore_barrier()

    @pl.loop(0, NCHA)
    def _(j):
        pltpu.async_copy(ones_v, acc_sh.at[idx_v.at[j]], sem, add=True)

    @pl.loop(0, NCHA)
    def _(j):
        pltpu.make_async_copy(ones_v, acc_sh.at[idx_v.at[0]], sem).wait()

    plsc.subcore_barrier()

    @pl.when(s == 0)
    def _():
        pltpu.sync_copy(acc_sh.at[pl.ds(0, N)], zb_v)
        pltpu.sync_copy(zb_v, deg_hbm.at[pl.ds(c * N, N)])


_degree_kernel = functools.partial(
    pl.kernel,
    out_type=jax.ShapeDtypeStruct((NC * N,), jnp.float32),
    mesh=_sc_mesh,
    scratch_types=[
        pltpu.VMEM((NCHA, KA), jnp.int32),
        pltpu.VMEM((KA,), jnp.float32),
        pltpu.VMEM((N,), jnp.float32),
        pltpu.VMEM_SHARED((N + 16,), jnp.float32),
        pltpu.SemaphoreType.DMA,
    ],
)(_degree_body)


def _matmul_body(x_ref, w_ref, deg_ref, out_ref):
    nsrc = lax.rsqrt(jnp.maximum(deg_ref[...], 1.0))
    h = jnp.dot(x_ref[...], w_ref[...],
                preferred_element_type=jnp.float32)
    h = h * nsrc
    out_ref[0] = h[:, :DH]
    out_ref[1] = h[:, DH:]


def _matmul_call(x, w, deg_out_col):
    return pl.pallas_call(
        _matmul_body,
        grid=(N // BLK,),
        in_specs=[
            pl.BlockSpec((BLK, D), lambda i: (i, 0)),
            pl.BlockSpec((D, D), lambda i: (0, 0)),
            pl.BlockSpec((BLK, 1), lambda i: (i, 0)),
        ],
        out_specs=pl.BlockSpec((2, BLK, DH), lambda i: (0, i, 0)),
        out_shape=jax.ShapeDtypeStruct((2, N, DH), jnp.float32),
    )(x, w, deg_out_col)


CK = 80
SCH = 16
EP = 10240
NCH = EP // CK
NPH = NCH // SCH
TRASH = N
WCH = 16
ACC_R = N + WCH
NZCH = ACC_R // WCH
NWCH = N // WCH


def _agg_body(srcmod, dst4, h2, out_hbm, sidx_v, didx_v, rows0, rows1,
              rows2, rows3, zb_v, acc_sh, sem0, sem1, sem2, sem3):
    c = lax.axis_index("c")
    s = lax.axis_index("s")

    @pl.loop(0, WCH)
    def _(i):
        for k in range(DH // L):
            zb_v[i, pl.ds(k * L, L)] = jnp.zeros((L,), jnp.float32)

    @pl.loop(s, NZCH, step=NS)
    def _(z):
        pltpu.sync_copy(zb_v, acc_sh.at[pl.ds(z * WCH, WCH)])

    plsc.subcore_barrier()

    @pl.loop(0, NPH)
    def _(ph):
        o = pl.multiple_of(ph * SCH, 8)
        pltpu.sync_copy(srcmod.at[c, s, pl.ds(o, SCH)], sidx_v)
        pltpu.sync_copy(dst4.at[s, pl.ds(o, SCH)], didx_v)

        for b, (rb, sb) in enumerate(
                ((rows0, sem0), (rows1, sem1), (rows2, sem2), (rows3, sem3))):
            pltpu.async_copy(h2.at[sidx_v.at[b]], rb, sb)

        @pl.loop(0, SCH // 4)
        def _(g):
            j0 = 4 * g
            for b, (rb, sb) in enumerate(
                    ((rows0, sem0), (rows1, sem1),
                     (rows2, sem2), (rows3, sem3))):
                j = j0 + b
                pltpu.make_async_copy(h2.at[sidx_v.at[j]], rb, sb).wait()
                pltpu.sync_copy(rb, acc_sh.at[didx_v.at[j]], add=True)

                @pl.when(j + 4 < SCH)
                def _():
                    pltpu.async_copy(h2.at[sidx_v.at[j + 4]], rb, sb)

    plsc.subcore_barrier()

    @pl.loop(s, NWCH, step=NS)
    def _(z):
        pltpu.sync_copy(acc_sh.at[pl.ds(z * WCH, WCH)], zb_v)
        pltpu.sync_copy(zb_v, out_hbm.at[c, pl.ds(z * WCH, WCH)])


_agg_kernel = functools.partial(
    pl.kernel,
    out_type=jax.ShapeDtypeStruct((NC, N, DH), jnp.float32),
    mesh=_sc_mesh,
    scratch_types=[
        pltpu.VMEM((SCH, CK), jnp.int32),
        pltpu.VMEM((SCH, CK), jnp.int32),
        pltpu.VMEM((CK, DH), jnp.float32),
        pltpu.VMEM((CK, DH), jnp.float32),
        pltpu.VMEM((CK, DH), jnp.float32),
        pltpu.VMEM((CK, DH), jnp.float32),
        pltpu.VMEM((WCH, DH), jnp.float32),
        pltpu.VMEM_SHARED((ACC_R, DH), jnp.float32),
        pltpu.SemaphoreType.DMA,
        pltpu.SemaphoreType.DMA,
        pltpu.SemaphoreType.DMA,
        pltpu.SemaphoreType.DMA,
    ],
)(_agg_body)


def _epilogue_body(agg_ref, deg_ref, b_ref, g_ref, be_ref, out_ref):
    ndst = lax.rsqrt(jnp.maximum(deg_ref[...], 1.0))
    inv = jnp.float32(1.0) / jnp.sqrt(jnp.float32(1.0 + EPS))
    for q in range(2):
        cols = slice(q * DH, (q + 1) * DH)
        y = agg_ref[q] * ndst + b_ref[:, cols]
        y = y * (g_ref[:, cols] * inv) + be_ref[:, cols]
        out_ref[:, cols] = jnp.maximum(y, 0.0)


def _epilogue_call(agg2, deg_in_col, b2, g2, be2):
    return pl.pallas_call(
        _epilogue_body,
        grid=(N // BLK,),
        in_specs=[
            pl.BlockSpec((2, BLK, DH), lambda i: (0, i, 0)),
            pl.BlockSpec((BLK, 1), lambda i: (i, 0)),
            pl.BlockSpec((1, D), lambda i: (0, 0)),
            pl.BlockSpec((1, D), lambda i: (0, 0)),
            pl.BlockSpec((1, D), lambda i: (0, 0)),
        ],
        out_specs=pl.BlockSpec((BLK, D), lambda i: (i, 0)),
        out_shape=jax.ShapeDtypeStruct((N, D), jnp.float32),
    )(agg2, deg_in_col, b2, g2, be2)


def kernel(node_features, edge_index, W, b, gamma, beta):
    src = edge_index[0]
    dst = edge_index[1]
    ept = E // NS
    srcp = jnp.pad(src.reshape(NS, ept), ((0, 0), (0, EP - ept)))
    dstp = jnp.pad(dst.reshape(NS, ept), ((0, 0), (0, EP - ept)),
                   constant_values=TRASH)
    srcpa = jnp.pad(src.reshape(NS, ept), ((0, 0), (0, EP - ept)),
                    constant_values=N)
    eidx_a = jnp.stack([srcpa, dstp]).reshape(NC, NS, NCHA, KA)
    srcmod = jnp.stack([srcp, srcp + N]).reshape(NC, NS, NCH, CK)
    dst4 = dstp.reshape(NS, NCH, CK)

    degs = _degree_kernel(eidx_a)
    deg_out_col = degs[:N].reshape(N, 1)
    deg_in_col = degs[N:].reshape(N, 1)

    h2 = _matmul_call(node_features, W, deg_out_col)

    agg2 = _agg_kernel(srcmod, dst4, h2.reshape(2 * N, DH))

    return _epilogue_call(agg2, deg_in_col, b.reshape(1, D),
                          gamma.reshape(1, D), beta.reshape(1, D))


# speedup vs baseline: 1.1002x; 1.0010x over previous
import functools

import jax
import jax.numpy as jnp
from jax import lax
from jax.experimental import pallas as pl
from jax.experimental.pallas import tpu as pltpu
from jax.experimental.pallas import tpu_sc as plsc

N = 10000
E = 160000
D = 256
DH = D // 2
NC, NS, L = 2, 16, 16
KA = 128
NCHA = 80
BLK = 1000
EPS = 1e-5

_sc_mesh = plsc.VectorSubcoreMesh(
    core_axis_name="c", subcore_axis_name="s", num_cores=NC, num_subcores=NS)


def _degree_body(eidx, deg_hbm, idx_v, ones_v, zb_v, acc_sh, sem):
    c = lax.axis_index("c")
    s = lax.axis_index("s")
    pltpu.sync_copy(eidx.at[c, s], idx_v)

    @pl.loop(0, KA // L)
    def _(i):
        ones_v[pl.ds(i * L, L)] = jnp.ones((L,), jnp.float32)

    @pl.when(s == 0)
    def _():
        @pl.loop(0, N // L)
        def _(i):
            zb_v[pl.ds(i * L, L)] = jnp.zeros((L,), jnp.float32)
        pltpu.sync_copy(zb_v, acc_sh.at[pl.ds(0, N)])
        pltpu.sync_copy(zb_v.at[pl.ds(0, 16)], acc_sh.at[pl.ds(N, 16)])

    plsc.subcore_barrier()

    @pl.loop(0, NCHA)
    def _(j):
        pltpu.async_copy(ones_v, acc_sh.at[idx_v.at[j]], sem, add=True)

    @pl.loop(0, NCHA)
    def _(j):
        pltpu.make_async_copy(ones_v, acc_sh.at[idx_v.at[0]], sem).wait()

    plsc.subcore_barrier()

    @pl.when(s == 0)
    def _():
        pltpu.sync_copy(acc_sh.at[pl.ds(0, N)], zb_v)
        pltpu.sync_copy(zb_v, deg_hbm.at[pl.ds(c * N, N)])


_degree_kernel = functools.partial(
    pl.kernel,
    out_type=jax.ShapeDtypeStruct((NC * N,), jnp.float32),
    mesh=_sc_mesh,
    scratch_types=[
        pltpu.VMEM((NCHA, KA), jnp.int32),
        pltpu.VMEM((KA,), jnp.float32),
        pltpu.VMEM((N,), jnp.float32),
        pltpu.VMEM_SHARED((N + 16,), jnp.float32),
        pltpu.SemaphoreType.DMA,
    ],
)(_degree_body)


def _matmul_body(x_ref, w_ref, deg_ref, out_ref):
    nsrc = lax.rsqrt(jnp.maximum(deg_ref[...], 1.0))
    h = jnp.dot(x_ref[...], w_ref[...],
                preferred_element_type=jnp.float32)
    h = h * nsrc
    out_ref[...] = h.reshape(BLK, 2, DH)


def _matmul_call(x, w, deg_out_col):
    return pl.pallas_call(
        _matmul_body,
        grid=(N // BLK,),
        in_specs=[
            pl.BlockSpec((BLK, D), lambda i: (i, 0)),
            pl.BlockSpec((D, D), lambda i: (0, 0)),
            pl.BlockSpec((BLK, 1), lambda i: (i, 0)),
        ],
        out_specs=pl.BlockSpec((BLK, 2, DH), lambda i: (i, 0, 0)),
        out_shape=jax.ShapeDtypeStruct((N, 2, DH), jnp.float32),
    )(x, w, deg_out_col)


CK = 80
SCH = 16
EP = 10240
NCH = EP // CK
NPH = NCH // SCH
TRASH = N
WCH = 16
ACC_R = N + WCH
NZCH = ACC_R // WCH
NWCH = N // WCH


def _agg_body(srcmod, dst4, h2, out_hbm, sidx_v, didx_v, rows0, rows1,
              rows2, rows3, zb_v, acc_sh, sem0, sem1, sem2, sem3):
    c = lax.axis_index("c")
    s = lax.axis_index("s")

    @pl.loop(0, WCH)
    def _(i):
        for k in range(DH // L):
            zb_v[i, pl.ds(k * L, L)] = jnp.zeros((L,), jnp.float32)

    @pl.loop(s, NZCH, step=NS)
    def _(z):
        pltpu.sync_copy(zb_v, acc_sh.at[pl.ds(z * WCH, WCH)])

    plsc.subcore_barrier()

    @pl.loop(0, NPH)
    def _(ph):
        o = pl.multiple_of(ph * SCH, 8)
        pltpu.sync_copy(srcmod.at[c, s, pl.ds(o, SCH)], sidx_v)
        pltpu.sync_copy(dst4.at[s, pl.ds(o, SCH)], didx_v)

        for b, (rb, sb) in enumerate(
                ((rows0, sem0), (rows1, sem1), (rows2, sem2), (rows3, sem3))):
            pltpu.async_copy(h2.at[sidx_v.at[b]], rb, sb)

        @pl.loop(0, SCH // 4)
        def _(g):
            j0 = 4 * g
            for b, (rb, sb) in enumerate(
                    ((rows0, sem0), (rows1, sem1),
                     (rows2, sem2), (rows3, sem3))):
                j = j0 + b
                pltpu.make_async_copy(h2.at[sidx_v.at[j]], rb, sb).wait()
                pltpu.sync_copy(rb, acc_sh.at[didx_v.at[j]], add=True)

                @pl.when(j + 4 < SCH)
                def _():
                    pltpu.async_copy(h2.at[sidx_v.at[j + 4]], rb, sb)

    plsc.subcore_barrier()

    @pl.loop(s, NWCH, step=NS)
    def _(z):
        pltpu.sync_copy(acc_sh.at[pl.ds(z * WCH, WCH)], zb_v)
        pltpu.sync_copy(zb_v, out_hbm.at[c, pl.ds(z * WCH, WCH)])


_agg_kernel = functools.partial(
    pl.kernel,
    out_type=jax.ShapeDtypeStruct((NC, N, DH), jnp.float32),
    mesh=_sc_mesh,
    scratch_types=[
        pltpu.VMEM((SCH, CK), jnp.int32),
        pltpu.VMEM((SCH, CK), jnp.int32),
        pltpu.VMEM((CK, DH), jnp.float32),
        pltpu.VMEM((CK, DH), jnp.float32),
        pltpu.VMEM((CK, DH), jnp.float32),
        pltpu.VMEM((CK, DH), jnp.float32),
        pltpu.VMEM((WCH, DH), jnp.float32),
        pltpu.VMEM_SHARED((ACC_R, DH), jnp.float32),
        pltpu.SemaphoreType.DMA,
        pltpu.SemaphoreType.DMA,
        pltpu.SemaphoreType.DMA,
        pltpu.SemaphoreType.DMA,
    ],
)(_agg_body)


def _epilogue_body(agg_ref, deg_ref, b_ref, g_ref, be_ref, out_ref):
    ndst = lax.rsqrt(jnp.maximum(deg_ref[...], 1.0))
    inv = jnp.float32(1.0) / jnp.sqrt(jnp.float32(1.0 + EPS))
    for q in range(2):
        cols = slice(q * DH, (q + 1) * DH)
        y = agg_ref[q] * ndst + b_ref[:, cols]
        y = y * (g_ref[:, cols] * inv) + be_ref[:, cols]
        out_ref[:, cols] = jnp.maximum(y, 0.0)


def _epilogue_call(agg2, deg_in_col, b2, g2, be2):
    return pl.pallas_call(
        _epilogue_body,
        grid=(N // BLK,),
        in_specs=[
            pl.BlockSpec((2, BLK, DH), lambda i: (0, i, 0)),
            pl.BlockSpec((BLK, 1), lambda i: (i, 0)),
            pl.BlockSpec((1, D), lambda i: (0, 0)),
            pl.BlockSpec((1, D), lambda i: (0, 0)),
            pl.BlockSpec((1, D), lambda i: (0, 0)),
        ],
        out_specs=pl.BlockSpec((BLK, D), lambda i: (i, 0)),
        out_shape=jax.ShapeDtypeStruct((N, D), jnp.float32),
    )(agg2, deg_in_col, b2, g2, be2)


def kernel(node_features, edge_index, W, b, gamma, beta):
    src = edge_index[0]
    dst = edge_index[1]
    ept = E // NS
    srcp = jnp.pad(src.reshape(NS, ept), ((0, 0), (0, EP - ept)))
    dstp = jnp.pad(dst.reshape(NS, ept), ((0, 0), (0, EP - ept)),
                   constant_values=TRASH)
    srcpa = jnp.pad(src.reshape(NS, ept), ((0, 0), (0, EP - ept)),
                    constant_values=N)
    eidx_a = jnp.stack([srcpa, dstp]).reshape(NC, NS, NCHA, KA)
    srcmod = jnp.stack([2 * srcp, 2 * srcp + 1]).reshape(NC, NS, NCH, CK)
    dst4 = dstp.reshape(NS, NCH, CK)

    degs = _degree_kernel(eidx_a)
    deg_out_col = degs[:N].reshape(N, 1)
    deg_in_col = degs[N:].reshape(N, 1)

    h2 = _matmul_call(node_features, W, deg_out_col)

    agg2 = _agg_kernel(srcmod, dst4, h2.reshape(2 * N, DH))

    return _epilogue_call(agg2, deg_in_col, b.reshape(1, D),
                          gamma.reshape(1, D), beta.reshape(1, D))
